# Initial kernel scaffold; baseline (speedup 1.0000x reference)
import jax
import jax.numpy as jnp
from jax.experimental import pallas as pl


def kernel(at_no, pos, edge_index, embed_table, W, b):
    raise NotImplementedError("write your pallas kernel here")


# speedup vs baseline: 1.2106x; 1.2106x over previous
import functools

import jax
import jax.numpy as jnp
import numpy as np
from jax import lax
from jax.experimental import pallas as pl
from jax.experimental.pallas import tpu as pltpu
from jax.experimental.pallas import tpu_sc as plsc

_N_NODES = 10000
_N_EDGES = 160000
_N_ELEM = 87
_EMBED_DIM = 28
_NODE_DIM = 128
_NUM_BASIS = 20
_CUTOFF = 5.0
_IRREPS = ((128, 0), (64, 1), (32, 2))
_RSH_DIM = 480

_NC, _NS = 2, 16
_NW = _NC * _NS
_LANES = 16

_NODES_PAD = 10240
_NPT = _NODES_PAD // _NW
_EDGES_PAD = 160256
_EPT = _EDGES_PAD // _NW

_EDGE_BLK = 640


def _make_expand_matrix():
    t = np.zeros((16, _RSH_DIM), dtype=np.float32)
    col = 0
    base = {0: 0, 1: 1, 2: 4}
    for mul, l in _IRREPS:
        width = 2 * l + 1
        for _ in range(mul):
            for k in range(width):
                t[base[l] + k, col] = 1.0
                col += 1
    assert col == _RSH_DIM
    return t


_EXPAND = _make_expand_matrix()


def _table_body(emb_ref, w_ref, b_ref, out_ref):
    out_ref[...] = lax.dot_general(
        emb_ref[...], w_ref[...],
        dimension_numbers=(((1,), (1,)), ((), ())),
        preferred_element_type=jnp.float32,
    ) + b_ref[...]


def _fused_table(embed_table, w, b):
    return pl.pallas_call(
        _table_body,
        out_shape=jax.ShapeDtypeStruct((_N_ELEM, _NODE_DIM), jnp.float32),
    )(embed_table, w, b.reshape(1, _NODE_DIM))


def _sc_body(table, at_no, px_h, py_h, pz_h, src_h, dst_h,
             xout, vx_h, vy_h, vz_h,
             idx_v, rows_v, px, py, pz, sv, dv, ox, oy, oz, sem):
    c = lax.axis_index("c")
    s = lax.axis_index("s")
    wid = s * _NC + c
    nbase = wid * _NPT
    ebase = wid * _EPT

    pltpu.sync_copy(at_no.at[pl.ds(nbase, _NPT)], idx_v)
    pltpu.async_copy(table.at[idx_v], rows_v, sem).wait()
    pltpu.sync_copy(rows_v, xout.at[pl.ds(nbase, _NPT)])

    pltpu.sync_copy(px_h, px)
    pltpu.sync_copy(py_h, py)
    pltpu.sync_copy(pz_h, pz)
    pltpu.sync_copy(src_h.at[pl.ds(ebase, _EPT)], sv)
    pltpu.sync_copy(dst_h.at[pl.ds(ebase, _EPT)], dv)

    def step(i, carry):
        o = i * _LANES
        si = sv[pl.ds(o, _LANES)]
        di = dv[pl.ds(o, _LANES)]
        ox[pl.ds(o, _LANES)] = (plsc.load_gather(px, [si])
                                - plsc.load_gather(px, [di]))
        oy[pl.ds(o, _LANES)] = (plsc.load_gather(py, [si])
                                - plsc.load_gather(py, [di]))
        oz[pl.ds(o, _LANES)] = (plsc.load_gather(pz, [si])
                                - plsc.load_gather(pz, [di]))
        return carry

    lax.fori_loop(0, _EPT // _LANES, step, 0)

    pltpu.sync_copy(ox, vx_h.at[pl.ds(ebase, _EPT)])
    pltpu.sync_copy(oy, vy_h.at[pl.ds(ebase, _EPT)])
    pltpu.sync_copy(oz, vz_h.at[pl.ds(ebase, _EPT)])


@functools.cache
def _sc_gather():
    return pl.kernel(
        _sc_body,
        out_type=[
            jax.ShapeDtypeStruct((_NODES_PAD, _NODE_DIM), jnp.float32),
            jax.ShapeDtypeStruct((_EDGES_PAD,), jnp.float32),
            jax.ShapeDtypeStruct((_EDGES_PAD,), jnp.float32),
            jax.ShapeDtypeStruct((_EDGES_PAD,), jnp.float32),
        ],
        mesh=plsc.VectorSubcoreMesh(
            core_axis_name="c", subcore_axis_name="s",
            num_cores=_NC, num_subcores=_NS),
        compiler_params=pltpu.CompilerParams(needs_layout_passes=False),
        scratch_types=[
            pltpu.VMEM((_NPT,), jnp.int32),
            pltpu.VMEM((_NPT, _NODE_DIM), jnp.float32),
            pltpu.VMEM((_NODES_PAD,), jnp.float32),
            pltpu.VMEM((_NODES_PAD,), jnp.float32),
            pltpu.VMEM((_NODES_PAD,), jnp.float32),
            pltpu.VMEM((_EPT,), jnp.int32),
            pltpu.VMEM((_EPT,), jnp.int32),
            pltpu.VMEM((_EPT,), jnp.float32),
            pltpu.VMEM((_EPT,), jnp.float32),
            pltpu.VMEM((_EPT,), jnp.float32),
            pltpu.SemaphoreType.DMA,
        ],
    )


def _edge_body(vx_ref, vy_ref, vz_ref, t_ref, rbf_ref, fcut_ref, rsh_ref):
    vx = vx_ref[...]
    vy = vy_ref[...]
    vz = vz_ref[...]
    r2 = vx * vx + vy * vy + vz * vz
    inv = lax.rsqrt(r2)
    dist = r2 * inv

    n = lax.broadcasted_iota(
        jnp.int32, (_EDGE_BLK, _NUM_BASIS), 1).astype(jnp.float32) + 1.0
    arg = (dist * (np.pi / _CUTOFF)) * n
    rbf_ref[...] = np.float32(np.sqrt(2.0 / _CUTOFF)) * jnp.sin(arg) * inv

    xr = dist * np.float32(1.0 / _CUTOFF)
    xr2 = xr * xr
    xr5 = xr2 * xr2 * xr
    fcut = 1.0 - 21.0 * xr5 + 35.0 * xr5 * xr - 15.0 * xr5 * xr2
    fcut_ref[...] = jnp.where(xr < 1.0, fcut, 0.0)

    ux = vx * inv
    uy = vy * inv
    uz = vz * inv
    s3 = np.float32(np.sqrt(3.0))
    s5 = np.float32(np.sqrt(5.0))
    s15 = np.float32(np.sqrt(15.0))
    cols = (
        jnp.ones_like(ux),
        s3 * ux,
        s3 * uy,
        s3 * uz,
        s15 * ux * uz,
        s15 * ux * uy,
        s5 * (uy * uy - 0.5 * (ux * ux + uz * uz)),
        s15 * uy * uz,
        (s15 * 0.5) * (uz * uz - ux * ux),
    )
    cidx = lax.broadcasted_iota(jnp.int32, (_EDGE_BLK, 16), 1)
    sh = jnp.zeros((_EDGE_BLK, 16), jnp.float32)
    for k, ck in enumerate(cols):
        sh = jnp.where(cidx == k, ck, sh)
    rsh_ref[...] = jnp.dot(sh, t_ref[...], preferred_element_type=jnp.float32)


def _edge_compute(vx, vy, vz, t):
    nblk = _N_EDGES // _EDGE_BLK
    return pl.pallas_call(
        _edge_body,
        grid=(nblk,),
        in_specs=[
            pl.BlockSpec((_EDGE_BLK, 1), lambda i: (i, 0)),
            pl.BlockSpec((_EDGE_BLK, 1), lambda i: (i, 0)),
            pl.BlockSpec((_EDGE_BLK, 1), lambda i: (i, 0)),
            pl.BlockSpec((16, _RSH_DIM), lambda i: (0, 0)),
        ],
        out_specs=[
            pl.BlockSpec((_EDGE_BLK, _NUM_BASIS), lambda i: (i, 0)),
            pl.BlockSpec((_EDGE_BLK, 1), lambda i: (i, 0)),
            pl.BlockSpec((_EDGE_BLK, _RSH_DIM), lambda i: (i, 0)),
        ],
        out_shape=[
            jax.ShapeDtypeStruct((_N_EDGES, _NUM_BASIS), jnp.float32),
            jax.ShapeDtypeStruct((_N_EDGES, 1), jnp.float32),
            jax.ShapeDtypeStruct((_N_EDGES, _RSH_DIM), jnp.float32),
        ],
    )(vx, vy, vz, t)


def kernel(at_no, pos, edge_index, embed_table, W, b):
    table = _fused_table(embed_table, W, b)

    at_no_pad = jnp.zeros((_NODES_PAD,), jnp.int32).at[:_N_NODES].set(
        at_no.astype(jnp.int32))
    src = jnp.zeros((_EDGES_PAD,), jnp.int32).at[:_N_EDGES].set(
        edge_index[0].astype(jnp.int32))
    dst = jnp.zeros((_EDGES_PAD,), jnp.int32).at[:_N_EDGES].set(
        edge_index[1].astype(jnp.int32))
    ppad = jnp.zeros((_NODES_PAD,), jnp.float32)
    px = ppad.at[:_N_NODES].set(pos[:, 1])
    py = ppad.at[:_N_NODES].set(pos[:, 2])
    pz = ppad.at[:_N_NODES].set(pos[:, 0])

    x_pad, vx, vy, vz = _sc_gather()(table, at_no_pad, px, py, pz, src, dst)

    rbf, fcut, rsh = _edge_compute(
        vx.reshape(_EDGES_PAD, 1),
        vy.reshape(_EDGES_PAD, 1),
        vz.reshape(_EDGES_PAD, 1),
        jnp.asarray(_EXPAND),
    )
    return (x_pad[:_N_NODES], rbf, fcut, rsh)
